# baseline (speedup 1.0000x reference)
import functools

import jax
import jax.numpy as jnp
import numpy as np
from jax import lax
from jax.experimental import pallas as pl
from jax.experimental.pallas import tpu as pltpu
from jax.experimental.pallas import tpu_sc as plsc

_HEADS = 16
_N = 32
_R = 2 * _N - 1
_L = 16
_NC = 2
_W0 = _HEADS * 64

_HH, _RR = np.meshgrid(np.arange(_HEADS), np.minimum(np.arange(64), _R - 1),
                       indexing="ij")
_IDX_H = (_RR * _HEADS + _HH).reshape(-1)
_IDX_W = ((_R - 1 - _RR) * _HEADS + _HH).reshape(-1)


def _bias_body(tab_hbm, out_hbm, tab_v, out_v, sem1, sem2):
    wid = lax.axis_index("s") * _NC + lax.axis_index("c")

    pltpu.sync_copy(tab_hbm, tab_v)

    def quarter(q):
        for h in range(q * 2, q * 2 + 2):
            for c in range(2):
                asc = tab_v[h, pl.ds(64 + 16 - 16 * c + wid, _L)]
                rv = lax.rev(asc, (0,))
                for i in range(_N):
                    bh = tab_v[h, pl.ds(16 * c + (_N - 1) - i, _L)]
                    out_v[h, i, pl.ds(16 * c, _L)] = bh + rv

    copies = []
    sems = [sem1, sem2]
    for q in range(8):
        quarter(q)
        copies.append(pltpu.async_copy(
            out_v.at[pl.ds(q * 2, 2)],
            out_hbm.at[wid, pl.ds(q * 2, 2)],
            sems[q % 2]))
    for cp in copies:
        cp.wait()


_bias_kernel = functools.partial(
    pl.kernel,
    mesh=plsc.VectorSubcoreMesh(core_axis_name="c", subcore_axis_name="s"),
    out_type=jax.ShapeDtypeStruct((_N, _HEADS, _N, _N), jnp.float32),
    scratch_types=[
        pltpu.VMEM((_HEADS, 128), jnp.float32),
        pltpu.VMEM((_HEADS, _N, _N), jnp.float32),
        pltpu.SemaphoreType.DMA,
        pltpu.SemaphoreType.DMA,
    ],
)(_bias_body)


def kernel(rel_height, rel_width, H, W):
    del H, W
    pad = jnp.zeros((_HEADS, 1), jnp.float32)
    tab = jnp.concatenate(
        [rel_height.T, pad, rel_width.T, pad], axis=1)
    return _bias_kernel(tab)


# speedup vs baseline: 1.0230x; 1.0230x over previous
import functools

import jax
import jax.numpy as jnp
import numpy as np
from jax import lax
from jax.experimental import pallas as pl
from jax.experimental.pallas import tpu as pltpu
from jax.experimental.pallas import tpu_sc as plsc

_HEADS = 16
_N = 32
_R = 2 * _N - 1
_L = 16
_NC = 2
_W0 = _HEADS * 64

_HH, _RR = np.meshgrid(np.arange(_HEADS), np.minimum(np.arange(64), _R - 1),
                       indexing="ij")
_IDX_H = (_RR * _HEADS + _HH).reshape(-1)
_IDX_W = ((_R - 1 - _RR) * _HEADS + _HH).reshape(-1)


def _bias_body(tab_hbm, out_hbm, tab_v, out_v, sem1, sem2):
    cid = lax.axis_index("c")
    wid = lax.axis_index("s") * _NC + cid

    pltpu.sync_copy(tab_hbm, tab_v)

    def units(kdyn, heads, vrows):
        for h, vr in zip(heads, vrows):
            for c in range(2):
                asc = tab_v[h, pl.ds(64 + 16 - 16 * c + kdyn, _L)]
                rv = lax.rev(asc, (0,))
                for i in range(_N):
                    bh = tab_v[h, pl.ds(16 * c + (_N - 1) - i, _L)]
                    out_v[vr, i, pl.ds(16 * c, _L)] = bh + rv

    copies = []
    sems = [sem1, sem2]
    chunks = [(0, 4), (4, 4), (8, 4), (12, 2)]
    for q, (lo, n) in enumerate(chunks):
        units(wid, range(lo, lo + n), range(lo, lo + n))
        copies.append(pltpu.async_copy(
            out_v.at[pl.ds(lo, n)],
            out_hbm.at[wid, pl.ds(lo, n)],
            sems[q % 2]))

    @pl.when(cid == 1)
    def _tail():
        units(wid, (14, 15), (14, 15))
        cp1 = pltpu.async_copy(
            out_v.at[pl.ds(14, 2)], out_hbm.at[wid, pl.ds(14, 2)], sem1)
        units(wid - 1, (14, 15), (16, 17))
        cp2 = pltpu.async_copy(
            out_v.at[pl.ds(16, 2)], out_hbm.at[wid - 1, pl.ds(14, 2)], sem2)
        cp1.wait()
        cp2.wait()

    for cp in copies:
        cp.wait()


_bias_kernel = functools.partial(
    pl.kernel,
    mesh=plsc.VectorSubcoreMesh(core_axis_name="c", subcore_axis_name="s"),
    out_type=jax.ShapeDtypeStruct((_N, _HEADS, _N, _N), jnp.float32),
    scratch_types=[
        pltpu.VMEM((_HEADS, 128), jnp.float32),
        pltpu.VMEM((_HEADS + 2, _N, _N), jnp.float32),
        pltpu.SemaphoreType.DMA,
        pltpu.SemaphoreType.DMA,
    ],
)(_bias_body)


def kernel(rel_height, rel_width, H, W):
    del H, W
    pad = jnp.zeros((_HEADS, 1), jnp.float32)
    tab = jnp.concatenate(
        [rel_height.T, pad, rel_width.T, pad], axis=1)
    return _bias_kernel(tab)


# speedup vs baseline: 1.0278x; 1.0047x over previous
import functools

import jax
import jax.numpy as jnp
from jax import lax
from jax.experimental import pallas as pl
from jax.experimental.pallas import tpu as pltpu
from jax.experimental.pallas import tpu_sc as plsc

_HEADS = 16
_N = 32
_L = 16
_NC = 2


def _bias_body(tab_hbm, out_hbm, tab_v, out_v, sem1, sem2):
    cid = lax.axis_index("c")
    wid = lax.axis_index("s") * _NC + cid

    pltpu.sync_copy(tab_hbm, tab_v)

    def units(kdyn, heads, vrows):
        for h, vr in zip(heads, vrows):
            for c in range(2):
                asc = tab_v[h, pl.ds(64 + 16 - 16 * c + kdyn, _L)]
                rv = lax.rev(asc, (0,))
                for i in range(_N):
                    bh = tab_v[h, pl.ds(16 * c + (_N - 1) - i, _L)]
                    out_v[vr, i, pl.ds(16 * c, _L)] = bh + rv

    copies = []
    sems = [sem1, sem2]
    chunks = [(0, 4), (4, 4), (8, 4), (12, 2)]
    for q, (lo, n) in enumerate(chunks):
        units(wid, range(lo, lo + n), range(lo, lo + n))
        copies.append(pltpu.async_copy(
            out_v.at[pl.ds(lo, n)],
            out_hbm.at[wid, pl.ds(lo, n)],
            sems[q % 2]))

    @pl.when(cid == 1)
    def _tail():
        units(wid, (14, 15), (14, 15))
        cp1 = pltpu.async_copy(
            out_v.at[pl.ds(14, 2)], out_hbm.at[wid, pl.ds(14, 2)], sem1)
        units(wid - 1, (14, 15), (16, 17))
        cp2 = pltpu.async_copy(
            out_v.at[pl.ds(16, 2)], out_hbm.at[wid - 1, pl.ds(14, 2)], sem2)
        cp1.wait()
        cp2.wait()

    for cp in copies:
        cp.wait()


_bias_kernel = functools.partial(
    pl.kernel,
    mesh=plsc.VectorSubcoreMesh(core_axis_name="c", subcore_axis_name="s"),
    out_type=jax.ShapeDtypeStruct((_N, _HEADS, _N, _N), jnp.float32),
    scratch_types=[
        pltpu.VMEM((_HEADS, 128), jnp.float32),
        pltpu.VMEM((_HEADS + 2, _N, _N), jnp.float32),
        pltpu.SemaphoreType.DMA,
        pltpu.SemaphoreType.DMA,
    ],
)(_bias_body)


def kernel(rel_height, rel_width, H, W):
    del H, W
    pad = jnp.zeros((_HEADS, 1), jnp.float32)
    tab = jnp.concatenate(
        [rel_height.T, pad, rel_width.T, pad], axis=1)
    return _bias_kernel(tab)
